# Initial kernel scaffold; baseline (speedup 1.0000x reference)
import jax
import jax.numpy as jnp
from jax.experimental import pallas as pl


def kernel(x, edge_index, adj_vals, W, prelu_a):
    raise NotImplementedError("write your pallas kernel here")


# speedup vs baseline: 6.4517x; 6.4517x over previous
import functools

import jax
import jax.numpy as jnp
from jax import lax
from jax.experimental import pallas as pl
from jax.experimental.pallas import tpu as pltpu
from jax.experimental.pallas import tpu_sc as plsc

N_NODES = 10000
N_EDGES = 320000
D = 128

NUM_WORKERS = 32
EDGES_PER_WORKER = N_EDGES // NUM_WORKERS
CHUNK = 80
SUPER = 25
NUM_SUPER = EDGES_PER_WORKER // (CHUNK * SUPER)
ROWS_PER_TILE = 632
N_PAD = 16 * ROWS_PER_TILE


def _mm_body(x_ref, w_ref, o_ref):
    o_ref[...] = jnp.dot(x_ref[...], w_ref[...],
                         preferred_element_type=jnp.float32)


def _matmul(x, w):
    blk = 2000
    return pl.pallas_call(
        _mm_body,
        grid=(N_NODES // blk,),
        in_specs=[
            pl.BlockSpec((blk, D), lambda i: (i, 0)),
            pl.BlockSpec((D, D), lambda i: (0, 0)),
        ],
        out_specs=pl.BlockSpec((blk, D), lambda i: (i, 0)),
        out_shape=jax.ShapeDtypeStruct((N_NODES, D), jnp.float32),
    )(x, w)


def _build_spmm():
    mesh = plsc.VectorSubcoreMesh(core_axis_name="c", subcore_axis_name="s")

    @functools.partial(
        pl.kernel,
        mesh=mesh,
        out_type=jax.ShapeDtypeStruct((2, N_PAD, D), jnp.float32),
        scratch_types=[
            pltpu.VMEM((SUPER, CHUNK), jnp.int32),
            pltpu.VMEM((SUPER, CHUNK), jnp.int32),
            pltpu.VMEM((SUPER * CHUNK,), jnp.float32),
            pltpu.VMEM((CHUNK, D), jnp.float32),
            pltpu.VMEM_SHARED((N_PAD, D), jnp.float32),
            pltpu.SemaphoreType.DMA,
        ],
    )
    def spmm(xh_hbm, col_hbm, row_hbm, adj_hbm, zero_hbm, out_hbm,
             col_v, row_v, adj_v, rows_v, acc_s, sem):
        c = lax.axis_index("c")
        s = lax.axis_index("s")
        wid = c * 16 + s

        pltpu.sync_copy(zero_hbm, acc_s.at[pl.ds(s * ROWS_PER_TILE,
                                                 ROWS_PER_TILE)])
        plsc.subcore_barrier()

        def super_body(u, carry):
            pltpu.sync_copy(col_hbm.at[wid, u], col_v)
            pltpu.sync_copy(row_hbm.at[wid, u], row_v)
            pltpu.sync_copy(adj_hbm.at[wid, u], adj_v)

            def chunk_body(j, carry2):
                pltpu.async_copy(xh_hbm.at[col_v.at[j]], rows_v, sem).wait()

                def scale_body(g, carry3):
                    av = adj_v[pl.ds(j * CHUNK + g * 16, 16)]
                    for e16 in range(16):
                        a = av[e16]
                        e = g * 16 + e16
                        for k in range(D // 16):
                            sl = pl.ds(k * 16, 16)
                            rows_v[e, sl] = rows_v[e, sl] * a
                    return carry3

                lax.fori_loop(0, CHUNK // 16, scale_body, 0, unroll=False)

                pltpu.sync_copy(rows_v, acc_s.at[row_v.at[j]], add=True)
                return carry2

            lax.fori_loop(0, SUPER, chunk_body, 0, unroll=False)
            return carry

        lax.fori_loop(0, NUM_SUPER, super_body, 0, unroll=False)
        plsc.subcore_barrier()

        sl = pl.ds(s * ROWS_PER_TILE, ROWS_PER_TILE)
        pltpu.sync_copy(acc_s.at[sl], out_hbm.at[c, sl])

    return spmm


_spmm = _build_spmm()


def _prelu_body(a_ref, p_ref, o_ref):
    a = a_ref[0, 0]
    agg = p_ref[0] + p_ref[1]
    o_ref[...] = jnp.where(agg >= 0.0, agg, a * agg)


def _prelu(parts, prelu_a):
    blk = 2000
    return pl.pallas_call(
        _prelu_body,
        grid=(N_NODES // blk,),
        in_specs=[
            pl.BlockSpec(memory_space=pltpu.SMEM),
            pl.BlockSpec((2, blk, D), lambda i: (0, i, 0)),
        ],
        out_specs=pl.BlockSpec((blk, D), lambda i: (i, 0)),
        out_shape=jax.ShapeDtypeStruct((N_NODES, D), jnp.float32),
    )(prelu_a, parts)


def kernel(x, edge_index, adj_vals, W, prelu_a):
    xh = _matmul(x, W)
    ei = edge_index.astype(jnp.int32)
    row4 = ei[0].reshape(NUM_WORKERS, NUM_SUPER, SUPER, CHUNK)
    col4 = ei[1].reshape(NUM_WORKERS, NUM_SUPER, SUPER, CHUNK)
    adj3 = adj_vals.reshape(NUM_WORKERS, NUM_SUPER, SUPER * CHUNK)
    zero_rows = jnp.zeros((ROWS_PER_TILE, D), jnp.float32)
    parts = _spmm(xh, col4, row4, adj3, zero_rows)
    return _prelu(parts, prelu_a.reshape(1, 1))
